# baseline (speedup 1.0000x reference)
import jax
import jax.numpy as jnp
from jax.experimental import pallas as pl
from jax.experimental.pallas import tpu as pltpu
from jax.experimental.pallas import tpu_sc as plsc

_W = 256
_L = 16
_TD = 8
_TB = 128


def _gather_tiled(w_rm, idx4, s, b, d):
    nb = b // _W
    ndg = d // _TD
    nbt = b // _TB
    bt_per_w = _W // _TB
    mesh = plsc.VectorSubcoreMesh(core_axis_name="core", subcore_axis_name="subcore")

    @pl.kernel(
        out_type=jax.ShapeDtypeStruct((s, ndg, nbt, _TD, _TB), w_rm.dtype),
        mesh=mesh,
        scratch_types=[pltpu.VMEM((_W, d), w_rm.dtype)],
        compiler_params=pltpu.CompilerParams(
            use_tc_tiling_on_sc=False, needs_layout_passes=False
        ),
    )
    def gather_kernel(w_hbm, i_hbm, o_hbm, scratch):
        def body(i_vmem, o_vmem):
            pltpu.sync_copy(w_hbm.at[i_vmem.at[0, 0, 0]], scratch)
            base = jax.lax.iota(jnp.int32, _L)
            rows = [
                [base + (bt * _TB + c * _L) for c in range(_TB // _L)]
                for bt in range(bt_per_w)
            ]

            for dd in range(d):
                dg = dd // _TD
                dr = dd % _TD
                cols = jnp.full((_L,), dd, jnp.int32)
                for bt in range(bt_per_w):
                    for c in range(_TB // _L):
                        o_vmem[0, dg, bt, dr, pl.ds(c * _L, _L)] = plsc.load_gather(
                            scratch, [rows[bt][c], cols]
                        )

        pltpu.emit_pipeline(
            body,
            grid=(s, nb),
            in_specs=[
                pl.BlockSpec((1, 1, 1, _W), index_map=lambda i, j: (i, j, 0, 0))
            ],
            out_specs=[
                pl.BlockSpec(
                    (1, ndg, bt_per_w, _TD, _TB),
                    index_map=lambda i, j: (i, 0, j, 0, 0),
                )
            ],
            core_axis_name=("core", "subcore"),
            dimension_semantics=(pltpu.PARALLEL, pltpu.PARALLEL),
        )(i_hbm, o_hbm)

    return gather_kernel(w_rm, idx4)


def kernel(x, weight):
    b, s = x.shape
    d = weight.shape[1]
    idx4 = x.T.reshape(s, b // _W, 1, _W).astype(jnp.int32)
    out5 = _gather_tiled(weight, idx4, s, b, d)
    return jnp.transpose(out5, (2, 4, 0, 1, 3)).reshape(b, s, d)


# speedup vs baseline: 1.6565x; 1.6565x over previous
import jax
import jax.numpy as jnp
from jax.experimental import pallas as pl
from jax.experimental.pallas import tpu as pltpu
from jax.experimental.pallas import tpu_sc as plsc

_W = 256
_TD = 8
_TB = 128
_OSLAB = 4


def _gather_rows(weight, idx2d, n, dim):
    mesh = plsc.VectorSubcoreMesh(core_axis_name="core", subcore_axis_name="subcore")

    @pl.kernel(
        out_type=jax.ShapeDtypeStruct((n, dim), weight.dtype),
        mesh=mesh,
        compiler_params=pltpu.CompilerParams(use_tc_tiling_on_sc=False),
    )
    def gather_kernel(w_hbm, i_hbm, o_hbm):
        def body(i_vmem, o_vmem):
            pltpu.sync_copy(w_hbm.at[i_vmem.at[0]], o_vmem)

        pltpu.emit_pipeline(
            body,
            grid=(n // _W,),
            in_specs=[pl.BlockSpec((1, _W), index_map=lambda i: (0, i))],
            out_specs=[pl.BlockSpec((_W, dim), index_map=lambda i: (i, 0))],
            core_axis_name=("core", "subcore"),
            dimension_semantics=(pltpu.PARALLEL,),
        )(i_hbm, o_hbm)

    return gather_kernel(weight, idx2d)


def _transpose_out(g, s, b, d):
    g3 = g.reshape(s, b, d)
    ndg = d // _TD
    nbt = b // _TB

    def body(in_ref, out_ref):
        for j in range(_OSLAB):
            t = in_ref[j].T
            for dg in range(ndg):
                out_ref[j, dg] = (
                    t[dg * _TD : (dg + 1) * _TD, :].reshape(_TD, nbt, _TB)
                ).swapaxes(0, 1)

    return pl.pallas_call(
        body,
        grid=(s // _OSLAB,),
        in_specs=[pl.BlockSpec((_OSLAB, b, d), lambda i: (i, 0, 0))],
        out_specs=pl.BlockSpec(
            (_OSLAB, ndg, nbt, _TD, _TB), lambda i: (i, 0, 0, 0, 0)
        ),
        out_shape=jax.ShapeDtypeStruct((s, ndg, nbt, _TD, _TB), g.dtype),
        compiler_params=pltpu.CompilerParams(dimension_semantics=("parallel",)),
    )(g3)


def kernel(x, weight):
    b, s = x.shape
    n = b * s
    d = weight.shape[1]
    idx2d = x.T.reshape(1, n).astype(jnp.int32)
    g = _gather_rows(weight, idx2d, n, d)
    out5 = _transpose_out(g, s, b, d)
    return jnp.transpose(out5, (2, 4, 0, 1, 3)).reshape(b, s, d)


# speedup vs baseline: 1.6578x; 1.0008x over previous
import jax
import jax.numpy as jnp
from jax.experimental import pallas as pl
from jax.experimental.pallas import tpu as pltpu
from jax.experimental.pallas import tpu_sc as plsc

_W = 256
_TD = 8
_TB = 128
_OSLAB = 8


def _gather_rows(weight, idx2d, n, dim):
    mesh = plsc.VectorSubcoreMesh(core_axis_name="core", subcore_axis_name="subcore")

    @pl.kernel(
        out_type=jax.ShapeDtypeStruct((n, dim), weight.dtype),
        mesh=mesh,
        compiler_params=pltpu.CompilerParams(use_tc_tiling_on_sc=False),
    )
    def gather_kernel(w_hbm, i_hbm, o_hbm):
        def body(i_vmem, o_vmem):
            pltpu.sync_copy(w_hbm.at[i_vmem.at[0]], o_vmem)

        pltpu.emit_pipeline(
            body,
            grid=(n // _W,),
            in_specs=[pl.BlockSpec((1, _W), index_map=lambda i: (0, i))],
            out_specs=[pl.BlockSpec((_W, dim), index_map=lambda i: (i, 0))],
            core_axis_name=("core", "subcore"),
            dimension_semantics=(pltpu.PARALLEL,),
        )(i_hbm, o_hbm)

    return gather_kernel(weight, idx2d)


def _transpose_out(g, s, b, d):
    g3 = g.reshape(s, b, d)
    ndg = d // _TD
    nbt = b // _TB

    def body(in_ref, out_ref):
        for j in range(_OSLAB):
            t = in_ref[j].T
            for dg in range(ndg):
                out_ref[j, dg] = (
                    t[dg * _TD : (dg + 1) * _TD, :].reshape(_TD, nbt, _TB)
                ).swapaxes(0, 1)

    return pl.pallas_call(
        body,
        grid=(s // _OSLAB,),
        in_specs=[pl.BlockSpec((_OSLAB, b, d), lambda i: (i, 0, 0))],
        out_specs=pl.BlockSpec(
            (_OSLAB, ndg, nbt, _TD, _TB), lambda i: (i, 0, 0, 0, 0)
        ),
        out_shape=jax.ShapeDtypeStruct((s, ndg, nbt, _TD, _TB), g.dtype),
        compiler_params=pltpu.CompilerParams(
            dimension_semantics=("parallel",),
            vmem_limit_bytes=56 * 1024 * 1024,
        ),
    )(g3)


def kernel(x, weight):
    b, s = x.shape
    n = b * s
    d = weight.shape[1]
    idx2d = x.T.reshape(1, n).astype(jnp.int32)
    g = _gather_rows(weight, idx2d, n, d)
    out5 = _transpose_out(g, s, b, d)
    return jnp.transpose(out5, (2, 4, 0, 1, 3)).reshape(b, s, d)


# speedup vs baseline: 1.7398x; 1.0495x over previous
import jax
import jax.numpy as jnp
from jax.experimental import pallas as pl
from jax.experimental.pallas import tpu as pltpu
from jax.experimental.pallas import tpu_sc as plsc

_W = 256
_TD = 8
_TB = 128
_OSLAB = 8


def _gather_rows(weight, idx2d, n, dim):
    mesh = plsc.VectorSubcoreMesh(core_axis_name="core", subcore_axis_name="subcore")

    @pl.kernel(
        out_type=jax.ShapeDtypeStruct((n, dim), weight.dtype),
        mesh=mesh,
        compiler_params=pltpu.CompilerParams(use_tc_tiling_on_sc=False),
    )
    def gather_kernel(w_hbm, i_hbm, o_hbm):
        def body(i_vmem, o_vmem):
            pltpu.sync_copy(w_hbm.at[i_vmem.at[0]], o_vmem)

        pltpu.emit_pipeline(
            body,
            grid=(n // _W,),
            in_specs=[pl.BlockSpec((1, _W), index_map=lambda i: (0, i))],
            out_specs=[pl.BlockSpec((_W, dim), index_map=lambda i: (i, 0))],
            core_axis_name=("core", "subcore"),
            dimension_semantics=(pltpu.PARALLEL,),
        )(i_hbm, o_hbm)

    return gather_kernel(weight, idx2d)


def _transpose_out(g, s, b, d):
    g3 = g.reshape(s, b, d)
    ndg = d // _TD
    nbt = b // _TB

    def body(in_ref, out_ref):
        for j in range(_OSLAB):
            t = in_ref[j].T
            for dg in range(ndg):
                out_ref[j, dg] = (
                    t[dg * _TD : (dg + 1) * _TD, :].reshape(_TD, nbt, _TB)
                ).swapaxes(0, 1)

    return pl.pallas_call(
        body,
        grid=(s // _OSLAB,),
        in_specs=[pl.BlockSpec((_OSLAB, b, d), lambda i: (i, 0, 0))],
        out_specs=pl.BlockSpec(
            (_OSLAB, ndg, nbt, _TD, _TB), lambda i: (i, 0, 0, 0, 0)
        ),
        out_shape=jax.ShapeDtypeStruct((s, ndg, nbt, _TD, _TB), g.dtype),
        compiler_params=pltpu.CompilerParams(
            dimension_semantics=("parallel",),
            vmem_limit_bytes=56 * 1024 * 1024,
        ),
    )(g3)


def kernel(x, weight):
    b, s = x.shape
    n = b * s
    d = weight.shape[1]
    idx2d = x.T.reshape(1, n).astype(jnp.int32)
    g = _gather_rows(weight, idx2d, n, d)
    return jnp.transpose(g.reshape(s, b, d), (1, 0, 2))
